# Initial kernel scaffold; baseline (speedup 1.0000x reference)
import jax
import jax.numpy as jnp
from jax.experimental import pallas as pl


def kernel(points, features_0, features_1, features_2, W_fc, b_fc):
    raise NotImplementedError("write your pallas kernel here")


# speedup vs baseline: 8.7879x; 8.7879x over previous
import functools

import jax
import jax.numpy as jnp
from jax import lax
from jax.experimental import pallas as pl
from jax.experimental.pallas import tpu as pltpu
from jax.experimental.pallas import tpu_sc as plsc

_NC, _NS = 2, 16
_NW = _NC * _NS
_N_PTS = 4096
_CH = 64
_CHUNK = _N_PTS // _NW
_LEVELS = ((7, 256, 256), (5, 128, 128), (3, 64, 64))
_D_TOT = sum(k * k for k, _, _ in _LEVELS)


def _sc_gather(px, py, t0, t1, t2):
    mesh = plsc.VectorSubcoreMesh(core_axis_name="c", subcore_axis_name="s")

    @functools.partial(
        pl.kernel,
        mesh=mesh,
        compiler_params=pltpu.CompilerParams(use_tc_tiling_on_sc=False),
        out_type=jax.ShapeDtypeStruct((_D_TOT, _N_PTS, _CH), jnp.float32),
        scratch_types=[
            pltpu.VMEM((_CHUNK,), jnp.float32),
            pltpu.VMEM((_CHUNK,), jnp.float32),
            pltpu.VMEM((_CHUNK,), jnp.float32),
            pltpu.VMEM((_CHUNK,), jnp.float32),
            pltpu.VMEM((_CHUNK,), jnp.int32),
            pltpu.VMEM((_CHUNK, _CH), jnp.float32),
            pltpu.SemaphoreType.DMA,
        ],
    )
    def k(px_hbm, py_hbm, t0_hbm, t1_hbm, t2_hbm, g_hbm,
          px_v, py_v, xf_v, yf_v, idx_v, rows_v, sem):
        wid = lax.axis_index("s") * _NC + lax.axis_index("c")
        base = wid * _CHUNK
        pltpu.sync_copy(px_hbm.at[pl.ds(base, _CHUNK)], px_v)
        pltpu.sync_copy(py_hbm.at[pl.ds(base, _CHUNK)], py_v)

        dbase = 0
        for tab, (kk, h, w) in zip((t0_hbm, t1_hbm, t2_hbm), _LEVELS):
            half = kk // 2
            wm1 = float(w - 1)
            hm1 = float(h - 1)
            for j in range(_CHUNK // 16):
                s = pl.ds(j * 16, 16)
                xf_v[s] = jnp.clip(px_v[s] * wm1, 0.0, wm1)
                yf_v[s] = jnp.clip(py_v[s] * hm1, 0.0, hm1)

            def tap_body(t, carry, tab=tab, kk=kk, h=h, w=w, half=half,
                         dbase=dbase, wm1=wm1, hm1=hm1):
                dyf = (t // kk - half).astype(jnp.float32)
                dxf = (t % kk - half).astype(jnp.float32)
                for j in range(_CHUNK // 16):
                    s = pl.ds(j * 16, 16)
                    tx = jnp.clip(xf_v[s] + dxf, 0.0, wm1).astype(jnp.int32)
                    ty = jnp.clip(yf_v[s] + dyf, 0.0, hm1).astype(jnp.int32)
                    idx_v[s] = ty * w + tx
                pltpu.async_copy(tab.at[idx_v], rows_v, sem).wait()
                pltpu.sync_copy(rows_v, g_hbm.at[dbase + t, pl.ds(base, _CHUNK), :])
                return carry

            lax.fori_loop(0, kk * kk, tap_body, 0)
            dbase += kk * kk

    return k(px, py, t0, t1, t2)


def _tc_project(g_flat, w_t, b2):
    blk = 2048
    grid = (g_flat.shape[1] // blk,)

    def body(g_ref, w_ref, b_ref, o_ref):
        acc = lax.dot_general(g_ref[...], w_ref[...],
                              (((0,), (0,)), ((), ())),
                              preferred_element_type=jnp.float32)
        o_ref[...] = acc + b_ref[...]

    return pl.pallas_call(
        body,
        grid=grid,
        in_specs=[
            pl.BlockSpec((_D_TOT, blk), lambda i: (0, i)),
            pl.BlockSpec((_D_TOT, 32), lambda i: (0, 0)),
            pl.BlockSpec((1, 32), lambda i: (0, 0)),
        ],
        out_specs=pl.BlockSpec((blk, 32), lambda i: (i, 0)),
        out_shape=jax.ShapeDtypeStruct((g_flat.shape[1], 32), jnp.float32),
    )(g_flat, w_t, b2)


def kernel(points, features_0, features_1, features_2, W_fc, b_fc):
    px = points[0, :, 0]
    py = points[0, :, 1]
    tables = []
    for feat, (_, h, w) in zip((features_0, features_1, features_2), _LEVELS):
        tables.append(jnp.transpose(feat[0].reshape(_CH, h * w)))
    g = _sc_gather(px, py, *tables)
    g_flat = g.reshape(_D_TOT, _N_PTS * _CH)
    proj = _tc_project(g_flat, jnp.transpose(W_fc), b_fc.reshape(1, 32))
    return proj.reshape(1, _CH, _N_PTS, W_fc.shape[0])
